# Initial kernel scaffold; baseline (speedup 1.0000x reference)
import functools

import jax
import jax.numpy as jnp
from jax import lax
from jax.experimental import pallas as pl
from jax.experimental.pallas import tpu as pltpu
from jax.experimental.pallas import tpu_sc as plsc

D = 128
L = 16
NC = 2
NS = 16
NW = NC * NS
CHUNK = 128


def _make_sc_call(e_pad):
    cpw = e_pad // (NW * CHUNK)
    mesh = plsc.VectorSubcoreMesh(core_axis_name="c", subcore_axis_name="s",
                                  num_cores=NC)

    @functools.partial(
        pl.kernel,
        mesh=mesh,
        out_type=jax.ShapeDtypeStruct((e_pad,), jnp.float32),
        scratch_types=[
            pltpu.VMEM((CHUNK,), jnp.int32),
            pltpu.VMEM((CHUNK,), jnp.int32),
            pltpu.VMEM((CHUNK, D), jnp.float32),
            pltpu.VMEM((CHUNK, D), jnp.float32),
            pltpu.VMEM((CHUNK,), jnp.float32),
            pltpu.SemaphoreType.DMA,
        ],
    )
    def sc_call(src_hbm, dst_hbm, xu_hbm, xc_hbm, out_hbm,
                src_v, dst_v, u_v, c_v, o_v, sem):
        wid = lax.axis_index("s") * NC + lax.axis_index("c")

        def chunk_body(ch, carry):
            base = pl.multiple_of((wid * cpw + ch) * CHUNK, CHUNK)
            pltpu.sync_copy(src_hbm.at[pl.ds(base, CHUNK)], src_v)
            pltpu.sync_copy(dst_hbm.at[pl.ds(base, CHUNK)], dst_v)
            cp_u = pltpu.async_copy(xu_hbm.at[src_v], u_v, sem)
            cp_c = pltpu.async_copy(xc_hbm.at[dst_v], c_v, sem)
            cp_u.wait()
            cp_c.wait()

            def group_body(g, carry2):
                rows = g * L + lax.iota(jnp.int32, L, 0)

                def k_body(kk, acc):
                    for dk in range(8):
                        cols = jnp.full((L,), kk * 8 + dk, jnp.int32)
                        uv = plsc.load_gather(u_v, [rows, cols])
                        cv = plsc.load_gather(c_v, [rows, cols])
                        acc = acc + uv * cv
                    return acc

                acc = lax.fori_loop(0, D // 8, k_body,
                                    jnp.zeros((L,), jnp.float32))
                o_v[pl.ds(g * L, L)] = acc
                return carry2

            lax.fori_loop(0, CHUNK // L, group_body, 0)
            pltpu.sync_copy(o_v, out_hbm.at[pl.ds(base, CHUNK)])
            return carry

        lax.fori_loop(0, cpw, chunk_body, 0)

    return sc_call


def kernel(x_user, x_content, edge_label_index):
    e = edge_label_index.shape[1]
    tile = NW * CHUNK
    e_pad = ((e + tile - 1) // tile) * tile
    src = edge_label_index[0]
    dst = edge_label_index[1]
    if e_pad != e:
        pad = jnp.zeros((e_pad - e,), jnp.int32)
        src = jnp.concatenate([src, pad])
        dst = jnp.concatenate([dst, pad])
    out = _make_sc_call(e_pad)(src, dst, x_user, x_content)
    return out[:e]


# speedup vs baseline: 1.0228x; 1.0228x over previous
import functools

import jax
import jax.numpy as jnp
from jax import lax
from jax.experimental import pallas as pl
from jax.experimental.pallas import tpu as pltpu
from jax.experimental.pallas import tpu_sc as plsc

D = 128
L = 16
NC = 2
NS = 16
NW = NC * NS
CHUNK = 128


def _make_sc_call(e_pad):
    cpw = e_pad // (NW * CHUNK)
    mesh = plsc.VectorSubcoreMesh(core_axis_name="c", subcore_axis_name="s",
                                  num_cores=NC)

    @functools.partial(
        pl.kernel,
        mesh=mesh,
        compiler_params=pltpu.CompilerParams(needs_layout_passes=False),
        out_type=jax.ShapeDtypeStruct((e_pad,), jnp.float32),
        scratch_types=[
            pltpu.VMEM((CHUNK,), jnp.int32),
            pltpu.VMEM((CHUNK,), jnp.int32),
            pltpu.VMEM((CHUNK, D), jnp.float32),
            pltpu.VMEM((CHUNK, D), jnp.float32),
            pltpu.VMEM((CHUNK,), jnp.float32),
            pltpu.SemaphoreType.DMA,
        ],
    )
    def sc_call(src_hbm, dst_hbm, xu_hbm, xc_hbm, out_hbm,
                src_v, dst_v, u_v, c_v, o_v, sem):
        wid = lax.axis_index("s") * NC + lax.axis_index("c")

        def chunk_body(ch, carry):
            base = pl.multiple_of((wid * cpw + ch) * CHUNK, CHUNK)
            pltpu.sync_copy(src_hbm.at[pl.ds(base, CHUNK)], src_v)
            pltpu.sync_copy(dst_hbm.at[pl.ds(base, CHUNK)], dst_v)
            cp_u = pltpu.async_copy(xu_hbm.at[src_v], u_v, sem)
            cp_c = pltpu.async_copy(xc_hbm.at[dst_v], c_v, sem)
            cp_u.wait()
            cp_c.wait()

            def group_body(g, carry2):
                rows = g * L + lax.iota(jnp.int32, L)

                def k_body(kk, acc):
                    for dk in range(8):
                        cols = jnp.full((L,), kk * 8 + dk, jnp.int32)
                        uv = plsc.load_gather(u_v, [rows, cols])
                        cv = plsc.load_gather(c_v, [rows, cols])
                        acc = acc + uv * cv
                    return acc

                acc = lax.fori_loop(0, D // 8, k_body,
                                    jnp.zeros((L,), jnp.float32))
                o_v[pl.ds(g * L, L)] = acc
                return carry2

            lax.fori_loop(0, CHUNK // L, group_body, 0)
            pltpu.sync_copy(o_v, out_hbm.at[pl.ds(base, CHUNK)])
            return carry

        lax.fori_loop(0, cpw, chunk_body, 0)

    return sc_call


def kernel(x_user, x_content, edge_label_index):
    e = edge_label_index.shape[1]
    tile = NW * CHUNK
    e_pad = ((e + tile - 1) // tile) * tile
    src = edge_label_index[0]
    dst = edge_label_index[1]
    if e_pad != e:
        pad = jnp.zeros((e_pad - e,), jnp.int32)
        src = jnp.concatenate([src, pad])
        dst = jnp.concatenate([dst, pad])
    out = _make_sc_call(e_pad)(src, dst, x_user, x_content)
    return out[:e]


# speedup vs baseline: 4.7623x; 4.6559x over previous
import functools

import jax
import jax.numpy as jnp
from jax import lax
from jax.experimental import pallas as pl
from jax.experimental.pallas import tpu as pltpu
from jax.experimental.pallas import tpu_sc as plsc

D = 128
W = D // 2
L = 16
NC = 2
NS = 16
NW = NC * NS
CHUNK = 128
NBUF = 2


def _make_sc_call(e_pad):
    cpw = e_pad // (NW * CHUNK)
    epw = cpw * CHUNK
    mesh = plsc.VectorSubcoreMesh(core_axis_name="c", subcore_axis_name="s",
                                  num_cores=NC)

    @functools.partial(
        pl.kernel,
        mesh=mesh,
        compiler_params=pltpu.CompilerParams(needs_layout_passes=False,
                                             use_tc_tiling_on_sc=False),
        out_type=jax.ShapeDtypeStruct((e_pad,), jnp.float32),
        scratch_types=[
            pltpu.VMEM((epw,), jnp.int32),
            pltpu.VMEM((epw,), jnp.int32),
            [pltpu.VMEM((CHUNK, W), jnp.int32) for _ in range(NBUF)],
            [pltpu.VMEM((CHUNK, W), jnp.int32) for _ in range(NBUF)],
            [pltpu.VMEM((CHUNK,), jnp.float32) for _ in range(NBUF)],
            [pltpu.SemaphoreType.DMA for _ in range(NBUF)],
            [pltpu.SemaphoreType.DMA for _ in range(NBUF)],
        ],
    )
    def sc_call(src_hbm, dst_hbm, xu_hbm, xc_hbm, out_hbm,
                src_v, dst_v, u_v, c_v, o_v, sems, osems):
        wid = lax.axis_index("s") * NC + lax.axis_index("c")
        e0 = pl.multiple_of(wid * epw, CHUNK)
        pltpu.sync_copy(src_hbm.at[pl.ds(e0, epw)], src_v)
        pltpu.sync_copy(dst_hbm.at[pl.ds(e0, epw)], dst_v)

        def issue(ch, b):
            off = pl.multiple_of(ch * CHUNK, CHUNK)
            pltpu.async_copy(xu_hbm.at[src_v.at[pl.ds(off, CHUNK)]],
                             u_v[b], sems[b])
            pltpu.async_copy(xc_hbm.at[dst_v.at[pl.ds(off, CHUNK)]],
                             c_v[b], sems[b])

        def wait_gather(b):
            pltpu.make_async_copy(
                xu_hbm.at[src_v.at[pl.ds(0, CHUNK)]], u_v[b], sems[b]).wait()
            pltpu.make_async_copy(
                xc_hbm.at[dst_v.at[pl.ds(0, CHUNK)]], c_v[b], sems[b]).wait()

        def wait_out(b):
            pltpu.make_async_copy(
                o_v[b], out_hbm.at[pl.ds(0, CHUNK)], osems[b]).wait()

        lane15 = lax.iota(jnp.int32, L) == (L - 1)

        def compute(b):
            ub, cb = u_v[b], c_v[b]

            def edge_body(e, carry):
                parts = []
                for t in range(W // L):
                    uw = plsc.bitcast(ub[e, pl.ds(t * L, L)], jnp.bfloat16)
                    cw = plsc.bitcast(cb[e, pl.ds(t * L, L)], jnp.bfloat16)
                    pa, pb = plsc.unpack(uw * cw,
                                         format=plsc.PackFormat.INTERLEAVED)
                    parts.append(pa + pb)
                while len(parts) > 1:
                    parts = [x + y for x, y in zip(parts[::2], parts[1::2])]
                s = plsc.cumsum(parts[0])
                plsc.store_scatter(o_v[b], [jnp.full((L,), e, jnp.int32)], s,
                                   mask=lane15)
                return carry

            lax.fori_loop(0, CHUNK, edge_body, 0, unroll=4)

        for b in range(NBUF):
            issue(b, b)

        def outer(it, carry):
            for b in range(NBUF):
                ch = it * NBUF + b
                wait_gather(b)

                @pl.when(it > 0)
                def _():
                    wait_out(b)

                compute(b)

                @pl.when(ch + NBUF < cpw)
                def _():
                    issue(ch + NBUF, b)

                pltpu.async_copy(
                    o_v[b],
                    out_hbm.at[pl.ds(pl.multiple_of(e0 + ch * CHUNK, CHUNK),
                                     CHUNK)],
                    osems[b])
            return carry

        lax.fori_loop(0, cpw // NBUF, outer, 0)
        for b in range(NBUF):
            wait_out(b)

    return sc_call


def kernel(x_user, x_content, edge_label_index):
    e = edge_label_index.shape[1]
    tile = NW * CHUNK * NBUF
    e_pad = ((e + tile - 1) // tile) * tile
    src = edge_label_index[0]
    dst = edge_label_index[1]
    if e_pad != e:
        pad = jnp.zeros((e_pad - e,), jnp.int32)
        src = jnp.concatenate([src, pad])
        dst = jnp.concatenate([dst, pad])
    n = x_user.shape[0]
    xu_p = jax.lax.bitcast_convert_type(
        x_user.astype(jnp.bfloat16).reshape(n, W, 2), jnp.int32)
    xc_p = jax.lax.bitcast_convert_type(
        x_content.astype(jnp.bfloat16).reshape(n, W, 2), jnp.int32)
    out = _make_sc_call(e_pad)(src, dst, xu_p, xc_p)
    return out[:e]
